# baseline (speedup 1.0000x reference)
import functools

import jax
import jax.numpy as jnp
from jax import lax
from jax.experimental import pallas as pl
from jax.experimental.pallas import tpu as pltpu
from jax.experimental.pallas import tpu_sc as plsc

K = 8
N = 16384
M = 8192
D = 64
NB = 8

QB = 512
CHUNK = 2560
STEP = 512
NSTARTS = (N - CHUNK) // STEP + 1
CPS = CHUNK // STEP

PADW = 128

QC = 512
RB = QC * K

_REAL_MAX = 1e10
_EXTRACTED = 1e30
_CONSUMED = 1e38


def _knn_body(ws_ref, wn_ref, coords_ref, coords_b_ref, pts_ref, pts_b_ref,
              qb_ref, idx_ref, d_scr, rv_ref):
    i = pl.program_id(0)
    px = pts_ref[:, 0:1]
    py = pts_ref[:, 1:2]
    pz = pts_ref[:, 2:3]
    ps = px * px + py * py + pz * pz
    pbx = pts_b_ref[:, 0:1]
    pby = pts_b_ref[:, 1:2]
    pbz = pts_b_ref[:, 2:3]
    qlo = qb_ref[:, 0:1]
    qhi = qb_ref[:, 1:2]

    rv_ref[...] = jnp.full((QB, K), _EXTRACTED, jnp.float32)
    idx_ref[...] = jnp.zeros((QB, K), jnp.int32)

    w0 = ws_ref[i]
    wn = wn_ref[i]

    def load_chunk(c, fresh_lo=None):
        cc = coords_ref[c]
        cx = cc[0:1, :]
        cy = cc[1:2, :]
        cz = cc[2:3, :]
        cs = cx * cx + cy * cy + cz * cz
        cb = coords_b_ref[c]
        dot = (pbx * cb[0:1, :] + pby * cb[1:2, :]
               + pbz * cb[2:3, :])
        d2 = ps + cs - 2.0 * dot
        gi = jax.lax.broadcasted_iota(jnp.int32, (QB, CHUNK), 1) + c * STEP
        valid = (gi >= qlo) & (gi < qhi)
        if fresh_lo is not None:
            valid = valid & (gi >= fresh_lo)
        d_scr[...] = jnp.where(valid, d2, jnp.float32(_REAL_MAX))
        return gi

    def extract8(gi):
        cand_v = []
        cand_i = []
        for _ in range(K):
            d = d_scr[...]
            m = jnp.min(d, axis=1, keepdims=True)
            eq = d == m
            am = jnp.min(jnp.where(eq, gi, jnp.int32(2**31 - 1)),
                         axis=1, keepdims=True)
            cand_v.append(m)
            cand_i.append(am)
            d_scr[...] = jnp.where(eq, jnp.float32(_EXTRACTED), d)
        return cand_v, cand_i

    gi0 = load_chunk(w0)
    cv0, ci0 = extract8(gi0)
    rv_ref[...] = jnp.concatenate(cv0, axis=1)
    idx_ref[...] = jnp.concatenate(ci0, axis=1)

    def chunk_body(t, carry):
        c = jnp.minimum(w0 + CPS * t, NSTARTS - 1)
        gi = load_chunk(c, fresh_lo=w0 * STEP + t * CHUNK)
        cand_v, cand_i = extract8(gi)
        allv = jnp.concatenate([rv_ref[...]] + cand_v, axis=1)
        alli = jnp.concatenate([idx_ref[...]] + cand_i, axis=1)
        pos = jax.lax.broadcasted_iota(jnp.int32, (QB, 2 * K), 1)
        new_v = []
        new_i = []
        for _ in range(K):
            m = jnp.min(allv, axis=1, keepdims=True)
            p = jnp.min(jnp.where(allv == m, pos, jnp.int32(2 * K)),
                        axis=1, keepdims=True)
            sel = pos == p
            new_v.append(m)
            new_i.append(jnp.min(jnp.where(sel, alli, jnp.int32(2**31 - 1)),
                                 axis=1, keepdims=True))
            allv = jnp.where(sel, jnp.float32(_CONSUMED), allv)
        rv_ref[...] = jnp.concatenate(new_v, axis=1)
        idx_ref[...] = jnp.concatenate(new_i, axis=1)
        return carry

    jax.lax.fori_loop(1, wn, chunk_body, 0)


def _knn_call(ws, wn, coords_c, coords_cb, pts, pts_b, qb):
    grid_spec = pltpu.PrefetchScalarGridSpec(
        num_scalar_prefetch=2,
        grid=(M // QB,),
        in_specs=[
            pl.BlockSpec((NSTARTS, 3, CHUNK), lambda i, ws, wn: (0, 0, 0)),
            pl.BlockSpec((NSTARTS, 3, CHUNK), lambda i, ws, wn: (0, 0, 0)),
            pl.BlockSpec((QB, 3), lambda i, ws, wn: (i, 0)),
            pl.BlockSpec((QB, 3), lambda i, ws, wn: (i, 0)),
            pl.BlockSpec((QB, 2), lambda i, ws, wn: (i, 0)),
        ],
        out_specs=pl.BlockSpec((QB, K), lambda i, ws, wn: (i, 0)),
        scratch_shapes=[
            pltpu.VMEM((QB, CHUNK), jnp.float32),
            pltpu.VMEM((QB, K), jnp.float32),
        ],
    )
    return pl.pallas_call(
        _knn_body,
        grid_spec=grid_spec,
        out_shape=jax.ShapeDtypeStruct((M, K), jnp.int32),
    )(ws, wn, coords_c, coords_cb, pts, pts_b, qb)


_NW = 32
_BPW = (M * K) // _NW
_GCH = 512


def _gather_call(idx_flat, table):
    mesh = plsc.VectorSubcoreMesh(core_axis_name="c", subcore_axis_name="s")

    @functools.partial(
        pl.kernel,
        mesh=mesh,
        out_type=jax.ShapeDtypeStruct((M * K, PADW), jnp.float32),
        scratch_types=[
            pltpu.VMEM((_GCH,), jnp.int32),
            pltpu.VMEM((_GCH, PADW), jnp.float32),
            pltpu.SemaphoreType.DMA,
        ],
    )
    def gather_kernel(idx_hbm, tab_hbm, out_hbm, idx_v, rows_v, sem):
        wid = lax.axis_index("s") * 2 + lax.axis_index("c")
        base = wid * _BPW
        for t in range(_BPW // _GCH):
            off = base + t * _GCH
            pltpu.sync_copy(idx_hbm.at[pl.ds(off, _GCH)], idx_v)
            pltpu.async_copy(tab_hbm.at[idx_v], rows_v, sem).wait()
            pltpu.sync_copy(rows_v, out_hbm.at[pl.ds(off, _GCH)])

    return gather_kernel(idx_flat, table)


def _mlp_body(g_ref, p_ref, w00, b00, w01, b01, wsc, w10, b10, w11, b11, out_ref):
    g = g_ref[...]
    p3 = p_ref[...]
    pr = jnp.broadcast_to(p3[:, None, :], (QC, K, 3)).reshape(RB, 3)
    h = jnp.concatenate([g[:, :D], g[:, D:D + 3] - pr, g[:, D + 3:]], axis=1)
    hp = jnp.maximum(h, 0.0)
    dot = functools.partial(
        lax.dot_general,
        dimension_numbers=(((1,), (0,)), ((), ())),
        preferred_element_type=jnp.float32,
    )
    net = dot(hp, w00[...]) + b00[...]
    dx = dot(jnp.maximum(net, 0.0), w01[...]) + b01[...]
    h1 = dot(h, wsc[...]) + dx
    net2 = dot(jnp.maximum(h1, 0.0), w10[...]) + b10[...]
    dx2 = dot(jnp.maximum(net2, 0.0), w11[...]) + b11[...]
    h2 = h1 + dx2
    out_ref[...] = jnp.max(h2.reshape(QC, K, D), axis=1)


def _mlp_call(g, pts, w00, b00, w01, b01, wsc, w10, b10, w11, b11):
    def c(shape):
        return pl.BlockSpec(shape, lambda i: tuple(0 for _ in shape))

    return pl.pallas_call(
        _mlp_body,
        grid=(M // QC,),
        in_specs=[
            pl.BlockSpec((RB, PADW), lambda i: (i, 0)),
            pl.BlockSpec((QC, 3), lambda i: (i, 0)),
            c((PADW, D)), c((1, D)), c((D, D)), c((1, D)),
            c((PADW, D)), c((D, D)), c((1, D)), c((D, D)), c((1, D)),
        ],
        out_specs=pl.BlockSpec((QC, D), lambda i: (i, 0)),
        out_shape=jax.ShapeDtypeStruct((M, D), jnp.float32),
    )(g, pts, w00, b00, w01, b01, wsc, w10, b10, w11, b11)


def kernel(feats, coords, batch_x, pts, batch_y, W0_fc0, b0_fc0, W0_fc1,
           b0_fc1, W0_sc, W1_fc0, b1_fc0, W1_fc1, b1_fc1):
    batch_x = batch_x.astype(jnp.int32)
    batch_y = batch_y.astype(jnp.int32)

    ids = jnp.arange(NB, dtype=jnp.int32)
    x_start = jnp.searchsorted(batch_x, ids, side="left").astype(jnp.int32)
    x_end = jnp.searchsorted(batch_x, ids, side="right").astype(jnp.int32)
    qlo = x_start[batch_y]
    qhi = x_end[batch_y]
    qb = jnp.stack([qlo, qhi], axis=1)

    by = batch_y.reshape(M // QB, QB)
    ws_elem = x_start[by[:, 0]]
    we_elem = x_end[by[:, -1]]
    ws = jnp.clip(ws_elem // STEP, 0, NSTARTS - 1).astype(jnp.int32)
    wn = jnp.clip(-((ws * STEP - we_elem) // CHUNK), 1,
                  -(-N // CHUNK)).astype(jnp.int32)

    coords_t = coords.T
    coords_c = jnp.stack([coords_t[:, s * STEP:s * STEP + CHUNK]
                          for s in range(NSTARTS)])
    coords_cb = jax.lax.reduce_precision(coords_c, 8, 7)
    pts_b = jax.lax.reduce_precision(pts, 8, 7)
    x_idx = _knn_call(ws, wn, coords_c, coords_cb, pts, pts_b, qb)

    table = jnp.pad(jnp.concatenate([feats, coords], axis=1),
                    ((0, 0), (0, PADW - D - 3)))
    g = _gather_call(x_idx.reshape(-1), table)

    w00 = jnp.pad(W0_fc0, ((0, PADW - D - 3), (0, 0)))
    wsc = jnp.pad(W0_sc, ((0, PADW - D - 3), (0, 0)))
    return _mlp_call(g, pts, w00, b0_fc0.reshape(1, D), W0_fc1,
                     b0_fc1.reshape(1, D), wsc, W1_fc0,
                     b1_fc0.reshape(1, D), W1_fc1, b1_fc1.reshape(1, D))


# speedup vs baseline: 1.1462x; 1.1462x over previous
import functools

import jax
import jax.numpy as jnp
from jax import lax
from jax.experimental import pallas as pl
from jax.experimental.pallas import tpu as pltpu
from jax.experimental.pallas import tpu_sc as plsc

K = 8
N = 16384
M = 8192
D = 64
NB = 8

QB = 256
CHUNK = 2560
STEP = 512
NSTARTS = (N - CHUNK) // STEP + 1
CPS = CHUNK // STEP

PADW = 128

QC = 512
RB = QC * K

_REAL_MAX = 1e10
_EXTRACTED = 1e30
_CONSUMED = 1e38


def _knn_body(ws_ref, wn_ref, cs_ref, coords_b_ref, ps_ref, pts_b_ref,
              qb_ref, idx_ref, d_scr, rv_ref):
    i = pl.program_id(0)
    ps = ps_ref[...]
    pbx = pts_b_ref[:, 0:1]
    pby = pts_b_ref[:, 1:2]
    pbz = pts_b_ref[:, 2:3]
    qlo = qb_ref[:, 0:1]
    qhi = qb_ref[:, 1:2]

    rv_ref[...] = jnp.full((QB, K), _EXTRACTED, jnp.float32)
    idx_ref[...] = jnp.zeros((QB, K), jnp.int32)

    w0 = ws_ref[i]
    wn = wn_ref[i]

    def load_chunk(c, fresh_lo=None):
        cs = cs_ref[c]
        cb = coords_b_ref[c]
        dot = (pbx * cb[0:1, :] + pby * cb[1:2, :]
               + pbz * cb[2:3, :])
        d2 = ps + cs - 2.0 * dot
        gi = jax.lax.broadcasted_iota(jnp.int32, (QB, CHUNK), 1) + c * STEP
        valid = (gi >= qlo) & (gi < qhi)
        if fresh_lo is not None:
            valid = valid & (gi >= fresh_lo)
        d_scr[...] = jnp.where(valid, d2, jnp.float32(_REAL_MAX))
        return gi

    def extract8(gi):
        cand_v = []
        cand_i = []
        for _ in range(K):
            d = d_scr[...]
            m = jnp.min(d, axis=1, keepdims=True)
            eq = d == m
            am = jnp.min(jnp.where(eq, gi, jnp.int32(2**31 - 1)),
                         axis=1, keepdims=True)
            cand_v.append(m)
            cand_i.append(am)
            d_scr[...] = jnp.where(eq, jnp.float32(_EXTRACTED), d)
        return cand_v, cand_i

    gi0 = load_chunk(w0)
    cv0, ci0 = extract8(gi0)
    rv_ref[...] = jnp.concatenate(cv0, axis=1)
    idx_ref[...] = jnp.concatenate(ci0, axis=1)

    def chunk_body(t, carry):
        c = jnp.minimum(w0 + CPS * t, NSTARTS - 1)
        gi = load_chunk(c, fresh_lo=w0 * STEP + t * CHUNK)
        cand_v, cand_i = extract8(gi)
        allv = jnp.concatenate([rv_ref[...]] + cand_v, axis=1)
        alli = jnp.concatenate([idx_ref[...]] + cand_i, axis=1)
        pos = jax.lax.broadcasted_iota(jnp.int32, (QB, 2 * K), 1)
        new_v = []
        new_i = []
        for _ in range(K):
            m = jnp.min(allv, axis=1, keepdims=True)
            p = jnp.min(jnp.where(allv == m, pos, jnp.int32(2 * K)),
                        axis=1, keepdims=True)
            sel = pos == p
            new_v.append(m)
            new_i.append(jnp.min(jnp.where(sel, alli, jnp.int32(2**31 - 1)),
                                 axis=1, keepdims=True))
            allv = jnp.where(sel, jnp.float32(_CONSUMED), allv)
        rv_ref[...] = jnp.concatenate(new_v, axis=1)
        idx_ref[...] = jnp.concatenate(new_i, axis=1)
        return carry

    jax.lax.fori_loop(1, wn, chunk_body, 0)


def _knn_call(ws, wn, cs_c, coords_cb, ps, pts_b, qb):
    grid_spec = pltpu.PrefetchScalarGridSpec(
        num_scalar_prefetch=2,
        grid=(M // QB,),
        in_specs=[
            pl.BlockSpec((NSTARTS, 1, CHUNK), lambda i, ws, wn: (0, 0, 0)),
            pl.BlockSpec((NSTARTS, 3, CHUNK), lambda i, ws, wn: (0, 0, 0)),
            pl.BlockSpec((QB, 1), lambda i, ws, wn: (i, 0)),
            pl.BlockSpec((QB, 3), lambda i, ws, wn: (i, 0)),
            pl.BlockSpec((QB, 2), lambda i, ws, wn: (i, 0)),
        ],
        out_specs=pl.BlockSpec((QB, K), lambda i, ws, wn: (i, 0)),
        scratch_shapes=[
            pltpu.VMEM((QB, CHUNK), jnp.float32),
            pltpu.VMEM((QB, K), jnp.float32),
        ],
    )
    return pl.pallas_call(
        _knn_body,
        grid_spec=grid_spec,
        out_shape=jax.ShapeDtypeStruct((M, K), jnp.int32),
    )(ws, wn, cs_c, coords_cb, ps, pts_b, qb)


_NW = 32
_BPW = (M * K) // _NW
_GCH = 512


def _gather_call(idx_flat, table):
    mesh = plsc.VectorSubcoreMesh(core_axis_name="c", subcore_axis_name="s")

    @functools.partial(
        pl.kernel,
        mesh=mesh,
        out_type=jax.ShapeDtypeStruct((M * K, PADW), jnp.float32),
        scratch_types=[
            pltpu.VMEM((_GCH,), jnp.int32),
            pltpu.VMEM((_GCH, PADW), jnp.float32),
            pltpu.SemaphoreType.DMA,
        ],
    )
    def gather_kernel(idx_hbm, tab_hbm, out_hbm, idx_v, rows_v, sem):
        wid = lax.axis_index("s") * 2 + lax.axis_index("c")
        base = wid * _BPW
        for t in range(_BPW // _GCH):
            off = base + t * _GCH
            pltpu.sync_copy(idx_hbm.at[pl.ds(off, _GCH)], idx_v)
            pltpu.async_copy(tab_hbm.at[idx_v], rows_v, sem).wait()
            pltpu.sync_copy(rows_v, out_hbm.at[pl.ds(off, _GCH)])

    return gather_kernel(idx_flat, table)


def _mlp_body(g_ref, p_ref, w00, b00, w01, b01, wsc, w10, b10, w11, b11, out_ref):
    g = g_ref[...]
    p3 = p_ref[...]
    pr = jnp.broadcast_to(p3[:, None, :], (QC, K, 3)).reshape(RB, 3)
    h = jnp.concatenate([g[:, :D], g[:, D:D + 3] - pr, g[:, D + 3:]], axis=1)
    hp = jnp.maximum(h, 0.0)
    dot = functools.partial(
        lax.dot_general,
        dimension_numbers=(((1,), (0,)), ((), ())),
        preferred_element_type=jnp.float32,
    )
    net = dot(hp, w00[...]) + b00[...]
    dx = dot(jnp.maximum(net, 0.0), w01[...]) + b01[...]
    h1 = dot(h, wsc[...]) + dx
    net2 = dot(jnp.maximum(h1, 0.0), w10[...]) + b10[...]
    dx2 = dot(jnp.maximum(net2, 0.0), w11[...]) + b11[...]
    h2 = h1 + dx2
    out_ref[...] = jnp.max(h2.reshape(QC, K, D), axis=1)


def _mlp_call(g, pts, w00, b00, w01, b01, wsc, w10, b10, w11, b11):
    def c(shape):
        return pl.BlockSpec(shape, lambda i: tuple(0 for _ in shape))

    return pl.pallas_call(
        _mlp_body,
        grid=(M // QC,),
        in_specs=[
            pl.BlockSpec((RB, PADW), lambda i: (i, 0)),
            pl.BlockSpec((QC, 3), lambda i: (i, 0)),
            c((PADW, D)), c((1, D)), c((D, D)), c((1, D)),
            c((PADW, D)), c((D, D)), c((1, D)), c((D, D)), c((1, D)),
        ],
        out_specs=pl.BlockSpec((QC, D), lambda i: (i, 0)),
        out_shape=jax.ShapeDtypeStruct((M, D), jnp.float32),
    )(g, pts, w00, b00, w01, b01, wsc, w10, b10, w11, b11)


def kernel(feats, coords, batch_x, pts, batch_y, W0_fc0, b0_fc0, W0_fc1,
           b0_fc1, W0_sc, W1_fc0, b1_fc0, W1_fc1, b1_fc1):
    batch_x = batch_x.astype(jnp.int32)
    batch_y = batch_y.astype(jnp.int32)

    ids = jnp.arange(NB, dtype=jnp.int32)
    x_start = jnp.searchsorted(batch_x, ids, side="left").astype(jnp.int32)
    x_end = jnp.searchsorted(batch_x, ids, side="right").astype(jnp.int32)
    qlo = x_start[batch_y]
    qhi = x_end[batch_y]
    qb = jnp.stack([qlo, qhi], axis=1)

    by = batch_y.reshape(M // QB, QB)
    ws_elem = x_start[by[:, 0]]
    we_elem = x_end[by[:, -1]]
    ws = jnp.clip(ws_elem // STEP, 0, NSTARTS - 1).astype(jnp.int32)
    wn = jnp.clip(-((ws * STEP - we_elem) // CHUNK), 1,
                  -(-N // CHUNK)).astype(jnp.int32)

    coords_t = coords.T
    css = jnp.sum(coords * coords, axis=1)
    cs_c = jnp.stack([css[None, s * STEP:s * STEP + CHUNK]
                      for s in range(NSTARTS)])
    coords_cb = jax.lax.reduce_precision(
        jnp.stack([coords_t[:, s * STEP:s * STEP + CHUNK]
                   for s in range(NSTARTS)]), 8, 7)
    ps = jnp.sum(pts * pts, axis=1, keepdims=True)
    pts_b = jax.lax.reduce_precision(pts, 8, 7)
    x_idx = _knn_call(ws, wn, cs_c, coords_cb, ps, pts_b, qb)

    table = jnp.pad(jnp.concatenate([feats, coords], axis=1),
                    ((0, 0), (0, PADW - D - 3)))
    g = _gather_call(x_idx.reshape(-1), table)

    w00 = jnp.pad(W0_fc0, ((0, PADW - D - 3), (0, 0)))
    wsc = jnp.pad(W0_sc, ((0, PADW - D - 3), (0, 0)))
    return _mlp_call(g, pts, w00, b0_fc0.reshape(1, D), W0_fc1,
                     b0_fc1.reshape(1, D), wsc, W1_fc0,
                     b1_fc0.reshape(1, D), W1_fc1, b1_fc1.reshape(1, D))
